# baseline (speedup 1.0000x reference)
import math

import jax
import jax.numpy as jnp
from jax.experimental import pallas as pl
from jax.experimental.pallas import tpu as pltpu

_EPS = 1e-5


def _fold(gamma, beta, mean, var):
    scale = gamma / jnp.sqrt(var + _EPS)
    bias = beta - mean * scale
    return scale.astype(jnp.float32), bias.astype(jnp.float32)


def _fused_block_kernel(xs_ref, wf_ref, sA_ref, bA_ref, w2_ref, s2_ref,
                        b2_ref, out_ref, a_ref, b_ref, buf_ref):
    Cout = out_ref.shape[1]
    Cin = xs_ref.shape[2] // 2
    P = a_ref.shape[0]
    Ho = math.isqrt(out_ref.shape[2])
    Wo = out_ref.shape[2] // Ho
    Sr = Wo + 2
    M = Ho * Sr
    G = (buf_ref.shape[0] - M) // 2

    a_ref[...] = jnp.zeros_like(a_ref)
    b_ref[...] = jnp.zeros_like(b_ref)
    for i in range(1, Ho + 1):
        a_ref[Sr * i + 1:Sr * i + 1 + Wo, :] = \
            xs_ref[0, pl.ds(Wo * (2 * i - 1), Wo), :].astype(jnp.bfloat16)
    for i in range(Ho):
        b_ref[Sr * i + 1:Sr * i + 1 + Wo, :] = \
            xs_ref[0, pl.ds(Wo * 2 * i, Wo), :].astype(jnp.bfloat16)

    def A(s):
        return a_ref[pl.ds(s, M), :]

    def B(s):
        return b_ref[pl.ds(s, M), :]

    patch1 = jnp.concatenate([
        A(1),
        B(1),
        A(Sr + 1),
        jnp.concatenate([A(0)[:, Cin:], B(0)[:, Cin:]], 1),
        jnp.concatenate([A(Sr)[:, Cin:], B(1)[:, :Cin]], 1),
    ], axis=1)

    y = jnp.dot(patch1, wf_ref[...], preferred_element_type=jnp.float32)
    y = y * sA_ref[...] + bA_ref[...]
    ident = y[:, Cout:]
    row = jax.lax.broadcasted_iota(jnp.int32, (M, Cout), 0)
    valid = (row % Sr) < (Sr - 2)
    main = jnp.where(valid, jnp.maximum(y[:, :Cout], 0.0),
                     0.0).astype(jnp.bfloat16)

    buf_ref[0:G, :] = jnp.zeros((G, Cout), jnp.bfloat16)
    buf_ref[G:G + M, :] = main
    buf_ref[G + M:, :] = jnp.zeros((G, Cout), jnp.bfloat16)

    patch2 = jnp.concatenate(
        [buf_ref[pl.ds(G + Sr * (kh - 1) + (kw - 1), M), :]
         for kh in range(3) for kw in range(3)], axis=1)

    y2 = jnp.dot(patch2, w2_ref[...], preferred_element_type=jnp.float32)
    y2 = y2 * s2_ref[...] + b2_ref[...] + ident
    y2 = jnp.maximum(y2, 0.0)

    y2t = jnp.transpose(y2, (1, 0))
    for oh in range(Ho):
        out_ref[0, :, oh * Wo:(oh + 1) * Wo] = \
            y2t[:, oh * Sr:oh * Sr + Wo]


def kernel(x, conv1_w, bn1_gamma, bn1_beta, bn1_mean, bn1_var, conv2_w,
           bn2_gamma, bn2_beta, bn2_mean, bn2_var, down_w, bn_down_gamma,
           bn_down_beta, bn_down_mean, bn_down_var):
    B, Cin, H, W = x.shape
    Cout = conv1_w.shape[0]
    Ho, Wo = H // 2, W // 2
    Sr = Wo + 2
    M = Ho * Sr
    P = (Ho + 2) * Sr
    G = -(-(Sr + 2) // 16) * 16

    xn = jnp.transpose(x, (0, 2, 3, 1))
    xs = xn.reshape(B, H * Wo, 2 * Cin)

    w1 = jnp.transpose(conv1_w, (2, 3, 1, 0)).reshape(9, Cin, Cout)
    s1, b1 = _fold(bn1_gamma, bn1_beta, bn1_mean, bn1_var)
    wd = jnp.transpose(down_w[:, :, 0, 0], (1, 0))
    sd, bd = _fold(bn_down_gamma, bn_down_beta, bn_down_mean, bn_down_var)
    w2m = jnp.transpose(conv2_w, (2, 3, 1, 0)).reshape(9 * Cout, Cout)
    s2, b2 = _fold(bn2_gamma, bn2_beta, bn2_mean, bn2_var)

    perm = [1, 2, 4, 5, 7, 8, 0, 3, 6]
    w1p = w1[jnp.array(perm)].reshape(9 * Cin, Cout)
    wf = jnp.zeros((10 * Cin, 2 * Cout), jnp.float32)
    wf = wf.at[:9 * Cin, :Cout].set(w1p)
    wf = wf.at[9 * Cin:, Cout:].set(wd)

    wf = wf.astype(jnp.bfloat16)
    w2m = w2m.astype(jnp.bfloat16)
    sA = jnp.concatenate([s1, sd])[None, :]
    bA = jnp.concatenate([b1, bd])[None, :]
    s2 = s2[None, :]
    b2 = b2[None, :]

    flops = 2 * B * Ho * Wo * Cout * (9 * Cin + Cin + 9 * Cout)
    bytes_acc = 4 * xs.size + 2 * wf.size + 2 * w2m.size + 4 * B * M * Cout

    out = pl.pallas_call(
        _fused_block_kernel,
        out_shape=jax.ShapeDtypeStruct((B, Cout, Ho * Wo), jnp.float32),
        grid=(B,),
        in_specs=[
            pl.BlockSpec((1, H * Wo, 2 * Cin), lambda b: (b, 0, 0)),
            pl.BlockSpec((10 * Cin, 2 * Cout), lambda b: (0, 0)),
            pl.BlockSpec((1, 2 * Cout), lambda b: (0, 0)),
            pl.BlockSpec((1, 2 * Cout), lambda b: (0, 0)),
            pl.BlockSpec((9 * Cout, Cout), lambda b: (0, 0)),
            pl.BlockSpec((1, Cout), lambda b: (0, 0)),
            pl.BlockSpec((1, Cout), lambda b: (0, 0)),
        ],
        out_specs=pl.BlockSpec((1, Cout, Ho * Wo), lambda b: (b, 0, 0)),
        scratch_shapes=[
            pltpu.VMEM((P, 2 * Cin), jnp.bfloat16),
            pltpu.VMEM((P, 2 * Cin), jnp.bfloat16),
            pltpu.VMEM((M + 2 * G, Cout), jnp.bfloat16),
        ],
        compiler_params=pltpu.CompilerParams(
            dimension_semantics=("parallel",),
            vmem_limit_bytes=64 * 1024 * 1024),
        cost_estimate=pl.CostEstimate(flops=flops, transcendentals=0,
                                      bytes_accessed=bytes_acc),
    )(xs, wf, sA, bA, w2m, s2, b2)

    return out.reshape(B, Cout, Ho, Wo)


# speedup vs baseline: 1.2074x; 1.2074x over previous
import math

import jax
import jax.numpy as jnp
from jax.experimental import pallas as pl
from jax.experimental.pallas import tpu as pltpu

_EPS = 1e-5


def _fold(gamma, beta, mean, var):
    scale = gamma / jnp.sqrt(var + _EPS)
    bias = beta - mean * scale
    return scale.astype(jnp.float32), bias.astype(jnp.float32)


def _fused_block_kernel(xs_ref, wf_ref, sA_ref, bA_ref, w2_ref, s2_ref,
                        b2_ref, out_ref, a_ref, b_ref, buf_ref):
    M, Cout = out_ref.shape[1], out_ref.shape[2]
    Cin = xs_ref.shape[2] // 2
    G = (buf_ref.shape[0] - M) // 2
    P = a_ref.shape[0]
    Sr = (P - M) // 2
    Wo = Sr - 2
    Ho = M // Sr

    a_ref[...] = jnp.zeros_like(a_ref)
    b_ref[...] = jnp.zeros_like(b_ref)
    for i in range(1, Ho + 1):
        a_ref[Sr * i + 1:Sr * i + 1 + Wo, :] = \
            xs_ref[0, pl.ds(Wo * (2 * i - 1), Wo), :].astype(jnp.bfloat16)
    for i in range(Ho):
        b_ref[Sr * i + 1:Sr * i + 1 + Wo, :] = \
            xs_ref[0, pl.ds(Wo * 2 * i, Wo), :].astype(jnp.bfloat16)

    def A(s):
        return a_ref[pl.ds(s, M), :]

    def B(s):
        return b_ref[pl.ds(s, M), :]

    patch1 = jnp.concatenate([
        A(1),
        B(1),
        A(Sr + 1),
        jnp.concatenate([A(0)[:, Cin:], B(0)[:, Cin:]], 1),
        jnp.concatenate([A(Sr)[:, Cin:], B(1)[:, :Cin]], 1),
    ], axis=1)

    y = jnp.dot(patch1, wf_ref[...], preferred_element_type=jnp.float32)
    y = y * sA_ref[...] + bA_ref[...]
    ident = y[:, Cout:]
    row = jax.lax.broadcasted_iota(jnp.int32, (M, Cout), 0)
    valid = (row % Sr) < (Sr - 2)
    main = jnp.where(valid, jnp.maximum(y[:, :Cout], 0.0),
                     0.0).astype(jnp.bfloat16)

    buf_ref[0:G, :] = jnp.zeros((G, Cout), jnp.bfloat16)
    buf_ref[G:G + M, :] = main
    buf_ref[G + M:, :] = jnp.zeros((G, Cout), jnp.bfloat16)

    patch2 = jnp.concatenate(
        [buf_ref[pl.ds(G + Sr * (kh - 1) + (kw - 1), M), :]
         for kh in range(3) for kw in range(3)], axis=1)

    y2 = jnp.dot(patch2, w2_ref[...], preferred_element_type=jnp.float32)
    y2 = y2 * s2_ref[...] + b2_ref[...] + ident
    out_ref[0] = jnp.maximum(y2, 0.0).astype(jnp.bfloat16)


def kernel(x, conv1_w, bn1_gamma, bn1_beta, bn1_mean, bn1_var, conv2_w,
           bn2_gamma, bn2_beta, bn2_mean, bn2_var, down_w, bn_down_gamma,
           bn_down_beta, bn_down_mean, bn_down_var):
    B, Cin, H, W = x.shape
    Cout = conv1_w.shape[0]
    Ho, Wo = H // 2, W // 2
    Sr = Wo + 2
    M = Ho * Sr
    P = (Ho + 2) * Sr
    G = -(-(Sr + 2) // 16) * 16

    xn = jnp.transpose(x, (0, 2, 3, 1))
    xs = xn.reshape(B, H * Wo, 2 * Cin)

    w1 = jnp.transpose(conv1_w, (2, 3, 1, 0)).reshape(9, Cin, Cout)
    s1, b1 = _fold(bn1_gamma, bn1_beta, bn1_mean, bn1_var)
    wd = jnp.transpose(down_w[:, :, 0, 0], (1, 0))
    sd, bd = _fold(bn_down_gamma, bn_down_beta, bn_down_mean, bn_down_var)
    w2m = jnp.transpose(conv2_w, (2, 3, 1, 0)).reshape(9 * Cout, Cout)
    s2, b2 = _fold(bn2_gamma, bn2_beta, bn2_mean, bn2_var)

    perm = [1, 2, 4, 5, 7, 8, 0, 3, 6]
    w1p = w1[jnp.array(perm)].reshape(9 * Cin, Cout)
    wf = jnp.zeros((10 * Cin, 2 * Cout), jnp.float32)
    wf = wf.at[:9 * Cin, :Cout].set(w1p)
    wf = wf.at[9 * Cin:, Cout:].set(wd)

    wf = wf.astype(jnp.bfloat16)
    w2m = w2m.astype(jnp.bfloat16)
    sA = jnp.concatenate([s1, sd])[None, :]
    bA = jnp.concatenate([b1, bd])[None, :]
    s2 = s2[None, :]
    b2 = b2[None, :]

    flops = 2 * B * Ho * Wo * Cout * (9 * Cin + Cin + 9 * Cout)
    bytes_acc = 4 * xs.size + 2 * wf.size + 2 * w2m.size + 4 * B * M * Cout

    out = pl.pallas_call(
        _fused_block_kernel,
        out_shape=jax.ShapeDtypeStruct((B, M, Cout), jnp.bfloat16),
        grid=(B,),
        in_specs=[
            pl.BlockSpec((1, H * Wo, 2 * Cin), lambda b: (b, 0, 0)),
            pl.BlockSpec((10 * Cin, 2 * Cout), lambda b: (0, 0)),
            pl.BlockSpec((1, 2 * Cout), lambda b: (0, 0)),
            pl.BlockSpec((1, 2 * Cout), lambda b: (0, 0)),
            pl.BlockSpec((9 * Cout, Cout), lambda b: (0, 0)),
            pl.BlockSpec((1, Cout), lambda b: (0, 0)),
            pl.BlockSpec((1, Cout), lambda b: (0, 0)),
        ],
        out_specs=pl.BlockSpec((1, M, Cout), lambda b: (b, 0, 0)),
        scratch_shapes=[
            pltpu.VMEM((P, 2 * Cin), jnp.bfloat16),
            pltpu.VMEM((P, 2 * Cin), jnp.bfloat16),
            pltpu.VMEM((M + 2 * G, Cout), jnp.bfloat16),
        ],
        compiler_params=pltpu.CompilerParams(
            dimension_semantics=("parallel",),
            vmem_limit_bytes=64 * 1024 * 1024),
        cost_estimate=pl.CostEstimate(flops=flops, transcendentals=0,
                                      bytes_accessed=bytes_acc),
    )(xs, wf, sA, bA, w2m, s2, b2)

    outs = out.reshape(B, Ho, Sr, Cout)[:, :, :Wo, :]
    return jnp.transpose(outs, (0, 3, 1, 2)).astype(jnp.float32)
